# baseline (speedup 1.0000x reference)
import functools

import jax
import jax.numpy as jnp
from jax.experimental import pallas as pl
from jax.experimental.pallas import tpu as pltpu

THETA_GLO = 1.15
DELTA = 5
GAMMA = 0.5
EPS = 1e-6

BLOCK_B = 512
OH_S = 256.0


def _loss_kernel(ya_ref, yp_ref, ypd_ref, ta_ref, tp_ref, out_ref,
                 cp_ref, ohp_ref, *, d, n_rows):
    i = pl.program_id(0)

    a = ya_ref[...]
    p = yp_ref[...]
    p_diag = ypd_ref[...]
    ta = ta_ref[...]
    tp = tp_ref[...]

    @pl.when(i == 0)
    def _():
        cp_ref[...] = (jnp.sum(p * p - (2.0 * EPS) * p, axis=1,
                               keepdims=True).T + d * EPS * EPS)
        iota_p = jax.lax.broadcasted_iota(jnp.int32, ohp_ref.shape, 1)
        ohp_ref[...] = jnp.where(iota_p == tp, OH_S, 0.0).astype(jnp.bfloat16)

    c_p = cp_ref[...]
    r_a = jnp.sum(a * a + (2.0 * EPS) * a, axis=1, keepdims=True)

    iota_a = jax.lax.broadcasted_iota(jnp.int32, a.shape, 1)
    oh_a = jnp.where(iota_a == ta, OH_S, 0.0).astype(jnp.bfloat16)

    cross_m = jax.lax.dot_general(
        -2.0 * a, p, (((1,), (1,)), ((), ())),
        preferred_element_type=jnp.float32)
    pen = jax.lax.dot_general(
        oh_a, ohp_ref[...], (((1,), (1,)), ((), ())),
        preferred_element_type=jnp.float32)

    e = (c_p + cross_m) + pen
    m = jnp.min(e, axis=1, keepdims=True) + r_a
    d_n = jnp.sqrt(jnp.maximum(m, 0.0))

    diff = a - p_diag + EPS
    d_p = jnp.sqrt(jnp.maximum(jnp.sum(diff * diff, axis=1, keepdims=True), 0.0))

    theta = GAMMA * (d_p + d_n) * 0.5 + (1.0 - GAMMA) * THETA_GLO
    scale = 2.0 * DELTA
    loss = -(jax.nn.log_sigmoid(scale * (theta - d_p))
             + jax.nn.log_sigmoid(scale * (d_n - theta))) / scale

    @pl.when(i == 0)
    def _():
        out_ref[...] = jnp.zeros((1, 1), jnp.float32)

    out_ref[...] += jnp.sum(loss, keepdims=True) / n_rows


def kernel(y_a, y_p, targets):
    b, d = y_a.shape
    targets = targets.astype(jnp.int32)
    t_row = targets.reshape(b, 1)
    grid = b // BLOCK_B

    out = pl.pallas_call(
        functools.partial(_loss_kernel, d=d, n_rows=b),
        grid=(grid,),
        in_specs=[
            pl.BlockSpec((BLOCK_B, d), lambda i: (i, 0)),
            pl.BlockSpec((b, d), lambda i: (0, 0)),
            pl.BlockSpec((BLOCK_B, d), lambda i: (i, 0)),
            pl.BlockSpec((BLOCK_B, 1), lambda i: (i, 0)),
            pl.BlockSpec((b, 1), lambda i: (0, 0)),
        ],
        out_specs=pl.BlockSpec((1, 1), lambda i: (0, 0)),
        out_shape=jax.ShapeDtypeStruct((1, 1), jnp.float32),
        scratch_shapes=[
            pltpu.VMEM((1, b), jnp.float32),
            pltpu.VMEM((b, d), jnp.bfloat16),
        ],
    )(y_a, y_p, y_p, t_row, t_row)

    return out[0, 0]


# speedup vs baseline: 1.4665x; 1.4665x over previous
import functools

import jax
import jax.numpy as jnp
from jax.experimental import pallas as pl
from jax.experimental.pallas import tpu as pltpu

THETA_GLO = 1.15
DELTA = 5
GAMMA = 0.5
EPS = 1e-6

BLOCK_B = 512
OH_S = 256.0
K_CAT = 256


def _loss_kernel(ya_full_ref, yp_ref, ya_ref, ypd_ref, t_ref, out_ref,
                 acat_ref, pcat_ref, *, d, n_rows):
    i = pl.program_id(0)

    a = ya_ref[...]
    p_diag = ypd_ref[...]

    @pl.when(i == 0)
    def _():
        p = yp_ref[...]
        a_full = ya_full_ref[...]
        t = t_ref[...]
        c_p = (jnp.sum(p * p - (2.0 * EPS) * p, axis=1, keepdims=True)
               + d * EPS * EPS)
        c_hi = c_p.astype(jnp.bfloat16).astype(jnp.float32)
        c_lo = c_p - c_hi
        iota = jax.lax.broadcasted_iota(jnp.int32, (p.shape[0], d), 1)
        oh_p = jnp.where(iota == t, OH_S, 0.0)
        oh_p = jnp.where(iota == 100, c_hi, oh_p)
        oh_p = jnp.where(iota == 101, c_lo, oh_p)
        pcat_ref[:, :d] = p.astype(jnp.bfloat16)
        pcat_ref[:, d:] = oh_p.astype(jnp.bfloat16)
        oh_a = jnp.where(iota == t, OH_S, 0.0)
        oh_a = jnp.where((iota == 100) | (iota == 101), 1.0, oh_a)
        acat_ref[:, :d] = (-2.0 * a_full).astype(jnp.bfloat16)
        acat_ref[:, d:] = oh_a.astype(jnp.bfloat16)

    a_cat = acat_ref[pl.ds(i * BLOCK_B, BLOCK_B), :]
    e = jax.lax.dot_general(
        a_cat, pcat_ref[...], (((1,), (1,)), ((), ())),
        preferred_element_type=jnp.float32)

    r_a = jnp.sum(a * a + (2.0 * EPS) * a, axis=1, keepdims=True)
    m = jnp.min(e, axis=1, keepdims=True) + r_a
    d_n = jnp.sqrt(jnp.maximum(m, 0.0))

    diff = a - p_diag + EPS
    d_p = jnp.sqrt(jnp.maximum(jnp.sum(diff * diff, axis=1, keepdims=True), 0.0))

    theta = GAMMA * (d_p + d_n) * 0.5 + (1.0 - GAMMA) * THETA_GLO
    scale = 2.0 * DELTA
    loss = -(jax.nn.log_sigmoid(scale * (theta - d_p))
             + jax.nn.log_sigmoid(scale * (d_n - theta))) / scale

    @pl.when(i == 0)
    def _():
        out_ref[...] = jnp.zeros((1, 1), jnp.float32)

    out_ref[...] += jnp.sum(loss, keepdims=True) / n_rows


def kernel(y_a, y_p, targets):
    b, d = y_a.shape
    targets = targets.astype(jnp.int32)
    t_row = targets.reshape(b, 1)
    grid = b // BLOCK_B

    out = pl.pallas_call(
        functools.partial(_loss_kernel, d=d, n_rows=b),
        grid=(grid,),
        in_specs=[
            pl.BlockSpec((b, d), lambda i: (0, 0)),
            pl.BlockSpec((b, d), lambda i: (0, 0)),
            pl.BlockSpec((BLOCK_B, d), lambda i: (i, 0)),
            pl.BlockSpec((BLOCK_B, d), lambda i: (i, 0)),
            pl.BlockSpec((b, 1), lambda i: (0, 0)),
        ],
        out_specs=pl.BlockSpec((1, 1), lambda i: (0, 0)),
        out_shape=jax.ShapeDtypeStruct((1, 1), jnp.float32),
        scratch_shapes=[
            pltpu.VMEM((b, K_CAT), jnp.bfloat16),
            pltpu.VMEM((b, K_CAT), jnp.bfloat16),
        ],
    )(y_a, y_p, y_a, y_p, t_row)

    return out[0, 0]
